# baseline (speedup 1.0000x reference)
import functools

import jax
import jax.numpy as jnp
from jax import lax
from jax.experimental import pallas as pl
from jax.experimental.pallas import tpu as pltpu


def _prologue_kernel(emb_ref, lnw_ref, lnb_ref, poolt_ref, st_ref, wbt_ref,
                     *, K, N):
    emb = emb_ref[...]
    mean = jnp.mean(emb, axis=-1, keepdims=True)
    cen = emb - mean
    var = jnp.mean(cen * cen, axis=-1, keepdims=True)
    e = cen * lax.rsqrt(var + 1e-12)
    e = e * lnw_ref[...] + lnb_ref[...]

    logits = lax.dot_general(e, e, (((1,), (1,)), ((), ())),
                             preferred_element_type=jnp.float32)
    elu = jnp.where(logits > 0.0, logits,
                    jnp.exp(jnp.minimum(logits, 0.0)) - 1.0)
    m = jnp.max(elu, axis=1, keepdims=True)
    p = jnp.exp(elu - m)
    s1t = p / jnp.sum(p, axis=1, keepdims=True)

    st_ref[:, 0:N] = s1t
    rows = lax.broadcasted_iota(jnp.int32, (N, N), 0)
    cols = lax.broadcasted_iota(jnp.int32, (N, N), 1)
    t_prev = (rows == cols).astype(jnp.float32)
    t_cur = s1t
    for k in range(2, K):
        t_new = 2.0 * jnp.dot(s1t, t_cur,
                              preferred_element_type=jnp.float32) - t_prev
        st_ref[:, (k - 1) * N:k * N] = t_new
        t_prev, t_cur = t_cur, t_new

    wbt_ref[...] = lax.dot_general(poolt_ref[...], e,
                                   (((1,), (1,)), ((), ())),
                                   preferred_element_type=jnp.float32)


def _gconv_kernel(st_ref, wbt_ref, x_ref, out_ref, g_ref,
                  *, I, O, K, TB, N, NM):
    kio = K * I * O

    @pl.when(pl.program_id(0) == 0)
    def _build_g():
        rows = lax.broadcasted_iota(jnp.int32, (N, N), 0)
        cols = lax.broadcasted_iota(jnp.int32, (N, N), 1)
        eye = (rows == cols).astype(jnp.float32)
        for o in range(NM):
            for i in range(I):
                g = eye * wbt_ref[i * O + o:i * O + o + 1, :]
                for k in range(1, K):
                    r = (k * I + i) * O + o
                    g = g + (st_ref[:, (k - 1) * N:k * N]
                             * wbt_ref[r:r + 1, :])
                g_ref[o * I + i] = g.astype(jnp.bfloat16)

    x3 = x_ref[...]
    xm = x3.reshape(I * TB, N)
    z = jnp.dot(xm, st_ref[...],
                preferred_element_type=jnp.float32)

    xh = [x3[i].astype(jnp.bfloat16) for i in range(I)]
    for o in range(NM):
        acc = jnp.dot(xh[0], g_ref[o * I],
                      preferred_element_type=jnp.float32)
        for i in range(1, I):
            acc = acc + jnp.dot(xh[i], g_ref[o * I + i],
                                preferred_element_type=jnp.float32)
        out_ref[o] = acc + wbt_ref[kio + o:kio + o + 1, :]

    LT = 128
    RT = 32
    for lt in range(0, N, LT):
        for rt in range(0, TB, RT):
            accs = [jnp.broadcast_to(
                wbt_ref[kio + o:kio + o + 1, lt:lt + LT], (RT, LT))
                for o in range(NM, O)]
            for i in range(I):
                xs = x3[i][rt:rt + RT, lt:lt + LT]
                for j, o in enumerate(range(NM, O)):
                    accs[j] = accs[j] + xs * wbt_ref[i * O + o:i * O + o + 1,
                                                     lt:lt + LT]
                for k in range(1, K):
                    zs = z[i * TB + rt:i * TB + rt + RT,
                           (k - 1) * N + lt:(k - 1) * N + lt + LT]
                    for j, o in enumerate(range(NM, O)):
                        r = (k * I + i) * O + o
                        accs[j] = accs[j] + zs * wbt_ref[r:r + 1, lt:lt + LT]
            for j, o in enumerate(range(NM, O)):
                out_ref[o, rt:rt + RT, lt:lt + LT] = accs[j]


def kernel(x, node_embeddings, ln_weight, ln_bias, weights_pool, bias_pool):
    B, N, I = x.shape
    D = node_embeddings.shape[1]
    K = weights_pool.shape[1]
    O = bias_pool.shape[1]
    KIO = K * I * O

    pool_t = jnp.concatenate(
        [weights_pool.reshape(D, KIO), bias_pool], axis=1).T
    lnw = ln_weight.reshape(1, D)
    lnb = ln_bias.reshape(1, D)

    st, wbt = pl.pallas_call(
        functools.partial(_prologue_kernel, K=K, N=N),
        out_shape=(jax.ShapeDtypeStruct((N, (K - 1) * N), jnp.float32),
                   jax.ShapeDtypeStruct((KIO + O, N), jnp.float32)),
        grid=(1,),
        in_specs=[
            pl.BlockSpec((N, D), lambda i: (0, 0)),
            pl.BlockSpec((1, D), lambda i: (0, 0)),
            pl.BlockSpec((1, D), lambda i: (0, 0)),
            pl.BlockSpec((KIO + O, D), lambda i: (0, 0)),
        ],
        out_specs=(pl.BlockSpec((N, (K - 1) * N), lambda i: (0, 0)),
                   pl.BlockSpec((KIO + O, N), lambda i: (0, 0))),
        compiler_params=pltpu.CompilerParams(
            dimension_semantics=("arbitrary",)),
    )(node_embeddings, lnw, lnb, pool_t)

    TB = next(t for t in (128, 64, 32, 16, 8, 4, 2, 1) if B % t == 0)
    NM = 4
    xl = jnp.transpose(x, (2, 0, 1))

    out_l = pl.pallas_call(
        functools.partial(_gconv_kernel, I=I, O=O, K=K, TB=TB, N=N, NM=NM),
        out_shape=jax.ShapeDtypeStruct((O, B, N), jnp.float32),
        grid=(B // TB,),
        in_specs=[
            pl.BlockSpec((N, (K - 1) * N), lambda t: (0, 0)),
            pl.BlockSpec((KIO + O, N), lambda t: (0, 0)),
            pl.BlockSpec((I, TB, N), lambda t: (0, t, 0)),
        ],
        out_specs=pl.BlockSpec((O, TB, N), lambda t: (0, t, 0)),
        scratch_shapes=[pltpu.VMEM((NM * I, N, N), jnp.bfloat16)],
        compiler_params=pltpu.CompilerParams(
            dimension_semantics=("arbitrary",)),
    )(st, wbt, xl)

    return jnp.transpose(out_l, (1, 2, 0))


# speedup vs baseline: 1.0950x; 1.0950x over previous
import functools

import jax
import jax.numpy as jnp
from jax import lax
from jax.experimental import pallas as pl
from jax.experimental.pallas import tpu as pltpu


def _prologue_kernel(emb_ref, lnw_ref, lnb_ref, poolt_ref, st_ref, wbt_ref,
                     *, K, N):
    emb = emb_ref[...]
    mean = jnp.mean(emb, axis=-1, keepdims=True)
    cen = emb - mean
    var = jnp.mean(cen * cen, axis=-1, keepdims=True)
    e = cen * lax.rsqrt(var + 1e-12)
    e = e * lnw_ref[...] + lnb_ref[...]

    logits = lax.dot_general(e, e, (((1,), (1,)), ((), ())),
                             preferred_element_type=jnp.float32)
    elu = jnp.where(logits > 0.0, logits,
                    jnp.exp(jnp.minimum(logits, 0.0)) - 1.0)
    m = jnp.max(elu, axis=1, keepdims=True)
    p = jnp.exp(elu - m)
    s1t = p / jnp.sum(p, axis=1, keepdims=True)

    st_ref[:, 0:N] = s1t
    rows = lax.broadcasted_iota(jnp.int32, (N, N), 0)
    cols = lax.broadcasted_iota(jnp.int32, (N, N), 1)
    t_prev = (rows == cols).astype(jnp.float32)
    t_cur = s1t
    for k in range(2, K):
        t_new = 2.0 * jnp.dot(s1t, t_cur,
                              preferred_element_type=jnp.float32) - t_prev
        st_ref[:, (k - 1) * N:k * N] = t_new
        t_prev, t_cur = t_cur, t_new

    wbt_ref[...] = lax.dot_general(poolt_ref[...], e,
                                   (((1,), (1,)), ((), ())),
                                   preferred_element_type=jnp.float32)


def _gconv_kernel(st_ref, wbt_ref, x_ref, out_ref, g_ref=None,
                  *, I, O, K, TB, N, NM=0):
    kio = K * I * O

    if NM:
        @pl.when(pl.program_id(0) == 0)
        def _build_g():
            rows = lax.broadcasted_iota(jnp.int32, (N, N), 0)
            cols = lax.broadcasted_iota(jnp.int32, (N, N), 1)
            eye = (rows == cols).astype(jnp.float32)
            for o in range(NM):
                for i in range(I):
                    g = eye * wbt_ref[i * O + o:i * O + o + 1, :]
                    for k in range(1, K):
                        r = (k * I + i) * O + o
                        g = g + (st_ref[:, (k - 1) * N:k * N]
                                 * wbt_ref[r:r + 1, :])
                    g_ref[o * I + i] = g.astype(jnp.bfloat16)

    x3 = x_ref[...]
    xm = x3.reshape(I * TB, N)
    z = jnp.dot(xm, st_ref[...],
                preferred_element_type=jnp.float32)

    if NM:
        xh = [x3[i].astype(jnp.bfloat16) for i in range(I)]
        for o in range(NM):
            acc = jnp.dot(xh[0], g_ref[o * I],
                          preferred_element_type=jnp.float32)
            for i in range(1, I):
                acc = acc + jnp.dot(xh[i], g_ref[o * I + i],
                                    preferred_element_type=jnp.float32)
            out_ref[o] = acc + wbt_ref[kio + o:kio + o + 1, :]

    LT = 128
    RT = 32
    for og in range(NM, O, 4):
        osl = range(og, min(og + 4, O))
        for lt in range(0, N, LT):
            for rt in range(0, TB, RT):
                accs = [jnp.broadcast_to(
                    wbt_ref[kio + o:kio + o + 1, lt:lt + LT], (RT, LT))
                    for o in osl]
                for i in range(I):
                    xs = x3[i][rt:rt + RT, lt:lt + LT]
                    for j, o in enumerate(osl):
                        accs[j] = accs[j] + xs * wbt_ref[
                            i * O + o:i * O + o + 1, lt:lt + LT]
                    for k in range(1, K):
                        zs = z[i * TB + rt:i * TB + rt + RT,
                               (k - 1) * N + lt:(k - 1) * N + lt + LT]
                        for j, o in enumerate(osl):
                            r = (k * I + i) * O + o
                            accs[j] = accs[j] + zs * wbt_ref[r:r + 1,
                                                             lt:lt + LT]
                for j, o in enumerate(osl):
                    out_ref[o, rt:rt + RT, lt:lt + LT] = accs[j]


def kernel(x, node_embeddings, ln_weight, ln_bias, weights_pool, bias_pool):
    B, N, I = x.shape
    D = node_embeddings.shape[1]
    K = weights_pool.shape[1]
    O = bias_pool.shape[1]
    KIO = K * I * O

    pool_t = jnp.concatenate(
        [weights_pool.reshape(D, KIO), bias_pool], axis=1).T
    lnw = ln_weight.reshape(1, D)
    lnb = ln_bias.reshape(1, D)

    st, wbt = pl.pallas_call(
        functools.partial(_prologue_kernel, K=K, N=N),
        out_shape=(jax.ShapeDtypeStruct((N, (K - 1) * N), jnp.float32),
                   jax.ShapeDtypeStruct((KIO + O, N), jnp.float32)),
        grid=(1,),
        in_specs=[
            pl.BlockSpec((N, D), lambda i: (0, 0)),
            pl.BlockSpec((1, D), lambda i: (0, 0)),
            pl.BlockSpec((1, D), lambda i: (0, 0)),
            pl.BlockSpec((KIO + O, D), lambda i: (0, 0)),
        ],
        out_specs=(pl.BlockSpec((N, (K - 1) * N), lambda i: (0, 0)),
                   pl.BlockSpec((KIO + O, N), lambda i: (0, 0))),
        compiler_params=pltpu.CompilerParams(
            dimension_semantics=("arbitrary",)),
    )(node_embeddings, lnw, lnb, pool_t)

    TB = next(t for t in (128, 64, 32, 16, 8, 4, 2, 1) if B % t == 0)
    NM = 0
    xl = jnp.transpose(x, (2, 0, 1))

    out_l = pl.pallas_call(
        functools.partial(_gconv_kernel, I=I, O=O, K=K, TB=TB, N=N, NM=NM),
        out_shape=jax.ShapeDtypeStruct((O, B, N), jnp.float32),
        grid=(B // TB,),
        in_specs=[
            pl.BlockSpec((N, (K - 1) * N), lambda t: (0, 0)),
            pl.BlockSpec((KIO + O, N), lambda t: (0, 0)),
            pl.BlockSpec((I, TB, N), lambda t: (0, t, 0)),
        ],
        out_specs=pl.BlockSpec((O, TB, N), lambda t: (0, t, 0)),
        scratch_shapes=([pltpu.VMEM((NM * I, N, N), jnp.bfloat16)]
                        if NM else []),
        compiler_params=pltpu.CompilerParams(
            dimension_semantics=("arbitrary",)),
    )(st, wbt, xl)

    return jnp.transpose(out_l, (1, 2, 0))
